# baseline (speedup 1.0000x reference)
import jax
import jax.numpy as jnp
from jax import lax
from jax.experimental import pallas as pl
from jax.experimental.pallas import tpu as pltpu
from jax.experimental.pallas import tpu_sc as plsc

_NUM_NODES = 200000
_H = 512
_W = 512
_BETA_F = 100.0

_PAD_NODES = 200704
_NC, _NS = 2, 16
_NW = _NC * _NS
_B = _H * _W
_BPW = _B // _NW


def _node_vals_body(t1, t2, t3, a1, a2, a3, lv, out):
    m = jnp.minimum(
        jnp.minimum(a1[...] - t1[0, 0], a2[...] - t2[0, 0]),
        a3[...] - t3[0, 0],
    )
    out[...] = jax.nn.sigmoid(_BETA_F * m) * lv[...]


def _gather_body(table, idx, out, idx_v, vals_v, sem):
    wid = lax.axis_index("s") * _NC + lax.axis_index("c")
    base = wid * _BPW
    pltpu.sync_copy(idx.at[pl.ds(base, _BPW)], idx_v)
    pltpu.async_copy(table.at[idx_v], vals_v, sem).wait()
    pltpu.sync_copy(vals_v, out.at[pl.ds(base, _BPW)])


def kernel(a_scaled_1, a_scaled_2, a_scaled_3, thr_1, thr_2, thr_3,
           node_levels, pixel_to_node):
    pad = _PAD_NODES - _NUM_NODES
    rows = _PAD_NODES // 128
    a1 = jnp.pad(a_scaled_1, (0, pad)).reshape(rows, 128)
    a2 = jnp.pad(a_scaled_2, (0, pad)).reshape(rows, 128)
    a3 = jnp.pad(a_scaled_3, (0, pad)).reshape(rows, 128)
    lv = jnp.pad(node_levels, (0, pad)).reshape(rows, 128)
    t1 = thr_1.reshape(1, 1)
    t2 = thr_2.reshape(1, 1)
    t3 = thr_3.reshape(1, 1)

    smem = pl.BlockSpec(memory_space=pltpu.SMEM)
    vmem = pl.BlockSpec(memory_space=pltpu.VMEM)
    node_vals = pl.pallas_call(
        _node_vals_body,
        out_shape=jax.ShapeDtypeStruct((rows, 128), jnp.float32),
        in_specs=[smem, smem, smem, vmem, vmem, vmem, vmem],
        out_specs=vmem,
    )(t1, t2, t3, a1, a2, a3, lv)
    table = node_vals.reshape(-1)

    gk = pl.kernel(
        _gather_body,
        out_type=jax.ShapeDtypeStruct((_B,), jnp.float32),
        mesh=plsc.VectorSubcoreMesh(core_axis_name="c", subcore_axis_name="s"),
        scratch_types=[
            pltpu.VMEM((_BPW,), jnp.int32),
            pltpu.VMEM((_BPW,), jnp.float32),
            pltpu.SemaphoreType.DMA,
        ],
    )
    y = gk(table, pixel_to_node.astype(jnp.int32))
    return y.reshape(_H, _W)
